# baseline (speedup 1.0000x reference)
import functools

import jax
import jax.numpy as jnp
from jax import lax
from jax.experimental import pallas as pl
from jax.experimental.pallas import tpu as pltpu
from jax.experimental.pallas import tpu_sc as plsc

_B = 16384
_N_OPS = 100000
_OP_EMB = 64
_LIN = 64
_OUT = _OP_EMB + _LIN


_TCOLS = 4096
_NBLK = 13
_PAIR = _NBLK * _TCOLS


def _stage_body(lo_ref, hi_ref, o_ref):
    o_ref[...] = jnp.concatenate([lo_ref[...].T, hi_ref[...].T], axis=-1)


def _stage_table(table_t):
    return pl.pallas_call(
        _stage_body,
        grid=(_NBLK,),
        in_specs=[
            pl.BlockSpec((_OP_EMB, _TCOLS), lambda j: (0, j)),
            pl.BlockSpec((_OP_EMB, _TCOLS),
                         lambda j: (0, jnp.minimum(j + _NBLK, 2 * _NBLK - 2))),
        ],
        out_specs=pl.BlockSpec((_TCOLS, 2 * _OP_EMB), lambda j: (j, 0)),
        out_shape=jax.ShapeDtypeStruct((_PAIR, 2 * _OP_EMB), jnp.float32),
    )(table_t, table_t)


_info = plsc.get_sparse_core_info()
_NC, _NS = _info.num_cores, _info.num_subcores
_NW = _NC * _NS
_BPW = _B // _NW
_CH = 128
_NCH = _BPW // _CH
_L = _info.num_lanes

_sc_mesh = plsc.VectorSubcoreMesh(core_axis_name="c", subcore_axis_name="s")


@functools.partial(
    pl.kernel,
    mesh=_sc_mesh,
    out_type=jax.ShapeDtypeStruct((_B, _OUT), jnp.float32),
    scratch_types=[
        pltpu.VMEM((_NCH, _CH), jnp.int32),
        pltpu.VMEM((_NCH, _CH), jnp.int32),
        pltpu.VMEM((_NCH, _CH, 2 * _OP_EMB), jnp.float32),
        pltpu.VMEM((2, _CH, _OUT), jnp.float32),
        pltpu.SemaphoreType.DMA,
        pltpu.SemaphoreType.DMA,
    ],
)
def _sc_gather(ids_hbm, offs_hbm, table2_hbm, lin_hbm, out_hbm, idx_v,
               off_v, raw_v, cmp_v, gsem, osem):
    wid = lax.axis_index("s") * _NC + lax.axis_index("c")
    base = wid * _BPW
    for c in range(_NCH):
        pltpu.sync_copy(ids_hbm.at[pl.ds(base + c * _CH, _CH)], idx_v.at[c])
        pltpu.sync_copy(offs_hbm.at[pl.ds(base + c * _CH, _CH)], off_v.at[c])
    gathers = [
        pltpu.async_copy(table2_hbm.at[idx_v.at[c]], raw_v.at[c], gsem)
        for c in range(_NCH)
    ]
    outs = []
    for c in range(_NCH):
        buf = c % 2
        if c >= 2:
            outs[c - 2].wait()
        pltpu.sync_copy(lin_hbm.at[pl.ds(base + c * _CH, _CH)],
                        cmp_v.at[buf])
        gathers[c].wait()
        def _grp(g, carry, c=c, buf=buf):
            offv = off_v[c, pl.ds(g * _L, _L)]
            rawg = raw_v.at[c, pl.ds(g * _L, _L)]
            cmpg = cmp_v.at[buf, pl.ds(g * _L, _L)]
            for lane in range(_L):
                o = offv[lane]
                for q in range(_OP_EMB // _L):
                    cmpg[lane, pl.ds(q * _L, _L)] = (
                        rawg[lane, pl.ds(o + q * _L, _L)])
            return carry

        lax.fori_loop(0, _CH // _L, _grp, jnp.int32(0), unroll=2)
        outs.append(pltpu.async_copy(
            cmp_v.at[buf], out_hbm.at[pl.ds(base + c * _CH, _CH)], osem))
    for cp in outs[-2:]:
        cp.wait()


_ROWS_PER_BLK = 2048


def _lin_body(st_ref, at_ref, wst_ref, bs_ref, wat_ref, ba_ref, o_ref):
    se = lax.dot_general(st_ref[...], wst_ref[...], (((0,), (0,)), ((), ())),
                         preferred_element_type=jnp.float32) + bs_ref[...]
    ae = lax.dot_general(at_ref[...], wat_ref[...], (((0,), (0,)), ((), ())),
                         preferred_element_type=jnp.float32) + ba_ref[...]
    o_ref[:, _OP_EMB:] = jnp.concatenate([se, ae], axis=-1)


def _linears(shapes_t, attrs_t, w_shape_t, b_shape, w_attr_t, b_attr):
    grid = (_B // _ROWS_PER_BLK,)
    return pl.pallas_call(
        _lin_body,
        grid=grid,
        in_specs=[
            pl.BlockSpec((shapes_t.shape[0], _ROWS_PER_BLK), lambda i: (0, i)),
            pl.BlockSpec((attrs_t.shape[0], _ROWS_PER_BLK), lambda i: (0, i)),
            pl.BlockSpec(w_shape_t.shape, lambda i: (0, 0)),
            pl.BlockSpec((1, b_shape.shape[1]), lambda i: (0, 0)),
            pl.BlockSpec(w_attr_t.shape, lambda i: (0, 0)),
            pl.BlockSpec((1, b_attr.shape[1]), lambda i: (0, 0)),
        ],
        out_specs=pl.BlockSpec((_ROWS_PER_BLK, _OUT), lambda i: (i, 0)),
        out_shape=jax.ShapeDtypeStruct((_B, _OUT), jnp.float32),
    )(shapes_t, attrs_t, w_shape_t, b_shape, w_attr_t, b_attr)


def kernel(op_ids, shapes, attrs, op_table, w_shape, b_shape, w_attr, b_attr):
    lin = _linears(shapes.T, attrs.T, w_shape.T, b_shape.reshape(1, -1),
                   w_attr.T, b_attr.reshape(1, -1))
    table2 = _stage_table(op_table.T)
    hi = op_ids >= _PAIR
    ids2 = jnp.where(hi, op_ids - _PAIR, op_ids)
    offs = jnp.where(hi, _OP_EMB, 0).astype(jnp.int32)
    return _sc_gather(ids2, offs, table2, lin)


# speedup vs baseline: 1.0954x; 1.0954x over previous
import functools

import jax
import jax.numpy as jnp
from jax import lax
from jax.experimental import pallas as pl
from jax.experimental.pallas import tpu as pltpu
from jax.experimental.pallas import tpu_sc as plsc

_B = 16384
_N_OPS = 100000
_OP_EMB = 64
_LIN = 64
_OUT = _OP_EMB + _LIN


_TCOLS = 3328
_NBLK = 16
_PAIR = _NBLK * _TCOLS
_LROWS = _B // _NBLK
_HI_CLAMP = 30


def _stage_lin_body(lo_ref, hi_ref, st_ref, at_ref, wst_ref, bs_ref,
                    wat_ref, ba_ref, stage_ref, lin_ref):
    stage_ref[...] = jnp.concatenate([lo_ref[...].T, hi_ref[...].T], axis=-1)
    se = lax.dot_general(st_ref[...], wst_ref[...], (((0,), (0,)), ((), ())),
                         preferred_element_type=jnp.float32) + bs_ref[...]
    ae = lax.dot_general(at_ref[...], wat_ref[...], (((0,), (0,)), ((), ())),
                         preferred_element_type=jnp.float32) + ba_ref[...]
    lin_ref[...] = jnp.concatenate([se, ae], axis=-1)


def _stage_and_linears(table_t, shapes_t, attrs_t, w_shape_t, b_shape,
                       w_attr_t, b_attr):
    return pl.pallas_call(
        _stage_lin_body,
        grid=(_NBLK,),
        in_specs=[
            pl.BlockSpec((_OP_EMB, _TCOLS), lambda j: (0, j)),
            pl.BlockSpec((_OP_EMB, _TCOLS),
                         lambda j: (0, jnp.minimum(j + _NBLK, _HI_CLAMP))),
            pl.BlockSpec((shapes_t.shape[0], _LROWS), lambda j: (0, j)),
            pl.BlockSpec((attrs_t.shape[0], _LROWS), lambda j: (0, j)),
            pl.BlockSpec(w_shape_t.shape, lambda j: (0, 0)),
            pl.BlockSpec((1, b_shape.shape[1]), lambda j: (0, 0)),
            pl.BlockSpec(w_attr_t.shape, lambda j: (0, 0)),
            pl.BlockSpec((1, b_attr.shape[1]), lambda j: (0, 0)),
        ],
        out_specs=[
            pl.BlockSpec((_TCOLS, 2 * _OP_EMB), lambda j: (j, 0)),
            pl.BlockSpec((_LROWS, _LIN), lambda j: (j, 0)),
        ],
        out_shape=[
            jax.ShapeDtypeStruct((_PAIR, 2 * _OP_EMB), jnp.float32),
            jax.ShapeDtypeStruct((_B, _LIN), jnp.float32),
        ],
    )(table_t, table_t, shapes_t, attrs_t, w_shape_t, b_shape, w_attr_t,
      b_attr)


_info = plsc.get_sparse_core_info()
_NC, _NS = _info.num_cores, _info.num_subcores
_NW = _NC * _NS
_BPW = _B // _NW
_CH = 128
_NCH = _BPW // _CH
_L = _info.num_lanes

_sc_mesh = plsc.VectorSubcoreMesh(core_axis_name="c", subcore_axis_name="s")


@functools.partial(
    pl.kernel,
    mesh=_sc_mesh,
    out_type=jax.ShapeDtypeStruct((_B, _OUT), jnp.float32),
    scratch_types=[
        pltpu.VMEM((_NCH, _CH), jnp.int32),
        pltpu.VMEM((_NCH, _CH), jnp.int32),
        pltpu.VMEM((_NCH, _CH, 2 * _OP_EMB), jnp.float32),
        pltpu.VMEM((2, _CH, _OUT), jnp.float32),
        pltpu.VMEM((_CH, _LIN), jnp.float32),
        pltpu.SemaphoreType.DMA,
        pltpu.SemaphoreType.DMA,
    ],
)
def _sc_gather(ids_hbm, offs_hbm, table2_hbm, lin_hbm, out_hbm, idx_v,
               off_v, raw_v, cmp_v, lin_v, gsem, osem):
    wid = lax.axis_index("s") * _NC + lax.axis_index("c")
    base = wid * _BPW
    for c in range(_NCH):
        pltpu.sync_copy(ids_hbm.at[pl.ds(base + c * _CH, _CH)], idx_v.at[c])
        pltpu.sync_copy(offs_hbm.at[pl.ds(base + c * _CH, _CH)], off_v.at[c])
    gathers = [
        pltpu.async_copy(table2_hbm.at[idx_v.at[c]], raw_v.at[c], gsem)
        for c in range(_NCH)
    ]
    outs = []
    for c in range(_NCH):
        buf = c % 2
        if c >= 2:
            outs[c - 2].wait()
        pltpu.sync_copy(lin_hbm.at[pl.ds(base + c * _CH, _CH)], lin_v)
        gathers[c].wait()
        def _grp(g, carry, c=c, buf=buf):
            offv = off_v[c, pl.ds(g * _L, _L)]
            rawg = raw_v.at[c, pl.ds(g * _L, _L)]
            cmpg = cmp_v.at[buf, pl.ds(g * _L, _L)]
            ling = lin_v.at[pl.ds(g * _L, _L)]
            for lane in range(_L):
                o = offv[lane]
                for q in range(_OP_EMB // _L):
                    cmpg[lane, pl.ds(q * _L, _L)] = (
                        rawg[lane, pl.ds(o + q * _L, _L)])
                for q in range(_LIN // _L):
                    cmpg[lane, pl.ds(_OP_EMB + q * _L, _L)] = (
                        ling[lane, pl.ds(q * _L, _L)])
            return carry

        lax.fori_loop(0, _CH // _L, _grp, jnp.int32(0))
        outs.append(pltpu.async_copy(
            cmp_v.at[buf], out_hbm.at[pl.ds(base + c * _CH, _CH)], osem))
    for cp in outs[-2:]:
        cp.wait()


def kernel(op_ids, shapes, attrs, op_table, w_shape, b_shape, w_attr, b_attr):
    table2, lin = _stage_and_linears(
        op_table.T, shapes.T, attrs.T, w_shape.T, b_shape.reshape(1, -1),
        w_attr.T, b_attr.reshape(1, -1))
    hi = op_ids >= _PAIR
    ids2 = jnp.where(hi, op_ids - _PAIR, op_ids)
    offs = jnp.where(hi, _OP_EMB, 0).astype(jnp.int32)
    return _sc_gather(ids2, offs, table2, lin)


# speedup vs baseline: 1.0977x; 1.0021x over previous
import functools

import jax
import jax.numpy as jnp
from jax import lax
from jax.experimental import pallas as pl
from jax.experimental.pallas import tpu as pltpu
from jax.experimental.pallas import tpu_sc as plsc

_B = 16384
_N_OPS = 100000
_OP_EMB = 64
_LIN = 64
_OUT = _OP_EMB + _LIN


_TCOLS = 3328
_NBLK = 16
_PAIR = _NBLK * _TCOLS
_LROWS = _B // _NBLK
_HI_CLAMP = 30


def _stage_lin_body(lo_ref, hi_ref, st_ref, at_ref, wst_ref, bs_ref,
                    wat_ref, ba_ref, stage_ref, lin_ref):
    stage_ref[...] = jnp.concatenate([lo_ref[...].T, hi_ref[...].T], axis=-1)
    se = lax.dot_general(st_ref[...], wst_ref[...], (((0,), (0,)), ((), ())),
                         preferred_element_type=jnp.float32) + bs_ref[...]
    ae = lax.dot_general(at_ref[...], wat_ref[...], (((0,), (0,)), ((), ())),
                         preferred_element_type=jnp.float32) + ba_ref[...]
    lin_ref[...] = jnp.concatenate([se, ae], axis=-1)


def _stage_and_linears(table_t, shapes_t, attrs_t, w_shape_t, b_shape,
                       w_attr_t, b_attr):
    return pl.pallas_call(
        _stage_lin_body,
        grid=(_NBLK,),
        in_specs=[
            pl.BlockSpec((_OP_EMB, _TCOLS), lambda j: (0, j)),
            pl.BlockSpec((_OP_EMB, _TCOLS),
                         lambda j: (0, jnp.minimum(j + _NBLK, _HI_CLAMP))),
            pl.BlockSpec((shapes_t.shape[0], _LROWS), lambda j: (0, j)),
            pl.BlockSpec((attrs_t.shape[0], _LROWS), lambda j: (0, j)),
            pl.BlockSpec(w_shape_t.shape, lambda j: (0, 0)),
            pl.BlockSpec((1, b_shape.shape[1]), lambda j: (0, 0)),
            pl.BlockSpec(w_attr_t.shape, lambda j: (0, 0)),
            pl.BlockSpec((1, b_attr.shape[1]), lambda j: (0, 0)),
        ],
        out_specs=[
            pl.BlockSpec((_TCOLS, 2 * _OP_EMB), lambda j: (j, 0)),
            pl.BlockSpec((_LROWS, _LIN), lambda j: (j, 0)),
        ],
        out_shape=[
            jax.ShapeDtypeStruct((_PAIR, 2 * _OP_EMB), jnp.float32),
            jax.ShapeDtypeStruct((_B, _LIN), jnp.float32),
        ],
    )(table_t, table_t, shapes_t, attrs_t, w_shape_t, b_shape, w_attr_t,
      b_attr)


_info = plsc.get_sparse_core_info()
_NC, _NS = _info.num_cores, _info.num_subcores
_NW = _NC * _NS
_BPW = _B // _NW
_CH = 128
_NCH = _BPW // _CH
_L = _info.num_lanes

_sc_mesh = plsc.VectorSubcoreMesh(core_axis_name="c", subcore_axis_name="s")


@functools.partial(
    pl.kernel,
    mesh=_sc_mesh,
    out_type=jax.ShapeDtypeStruct((_B, _OUT), jnp.float32),
    scratch_types=[
        pltpu.VMEM((_BPW,), jnp.int32),
        pltpu.VMEM((_BPW,), jnp.int32),
        pltpu.VMEM((_NCH, _CH, 2 * _OP_EMB), jnp.float32),
        pltpu.VMEM((2, _CH, _OUT), jnp.float32),
        pltpu.VMEM((_CH, _LIN), jnp.float32),
        pltpu.SemaphoreType.DMA,
        pltpu.SemaphoreType.DMA,
    ],
)
def _sc_gather(ids_hbm, offs_hbm, table2_hbm, lin_hbm, out_hbm, idx_v,
               off_v, raw_v, cmp_v, lin_v, gsem, osem):
    wid = lax.axis_index("s") * _NC + lax.axis_index("c")
    base = wid * _BPW
    pltpu.sync_copy(ids_hbm.at[pl.ds(base, _BPW)], idx_v)
    pltpu.sync_copy(offs_hbm.at[pl.ds(base, _BPW)], off_v)
    gathers = [
        pltpu.async_copy(table2_hbm.at[idx_v.at[pl.ds(c * _CH, _CH)]],
                         raw_v.at[c], gsem)
        for c in range(_NCH)
    ]
    outs = []
    for c in range(_NCH):
        buf = c % 2
        if c >= 2:
            outs[c - 2].wait()
        pltpu.sync_copy(lin_hbm.at[pl.ds(base + c * _CH, _CH)], lin_v)
        gathers[c].wait()
        def _grp(g, carry, c=c, buf=buf):
            offv = off_v[pl.ds(c * _CH + g * _L, _L)]
            rawg = raw_v.at[c, pl.ds(g * _L, _L)]
            cmpg = cmp_v.at[buf, pl.ds(g * _L, _L)]
            ling = lin_v.at[pl.ds(g * _L, _L)]
            for lane in range(_L):
                o = offv[lane]
                for q in range(_OP_EMB // _L):
                    cmpg[lane, pl.ds(q * _L, _L)] = (
                        rawg[lane, pl.ds(o + q * _L, _L)])
                for q in range(_LIN // _L):
                    cmpg[lane, pl.ds(_OP_EMB + q * _L, _L)] = (
                        ling[lane, pl.ds(q * _L, _L)])
            return carry

        lax.fori_loop(0, _CH // _L, _grp, jnp.int32(0), unroll=2)
        outs.append(pltpu.async_copy(
            cmp_v.at[buf], out_hbm.at[pl.ds(base + c * _CH, _CH)], osem))
    for cp in outs[-2:]:
        cp.wait()


def kernel(op_ids, shapes, attrs, op_table, w_shape, b_shape, w_attr, b_attr):
    table2, lin = _stage_and_linears(
        op_table.T, shapes.T, attrs.T, w_shape.T, b_shape.reshape(1, -1),
        w_attr.T, b_attr.reshape(1, -1))
    hi = op_ids >= _PAIR
    ids2 = jnp.where(hi, op_ids - _PAIR, op_ids)
    offs = jnp.where(hi, _OP_EMB, 0).astype(jnp.int32)
    return _sc_gather(ids2, offs, table2, lin)


# speedup vs baseline: 1.1382x; 1.0369x over previous
import functools

import jax
import jax.numpy as jnp
from jax import lax
from jax.experimental import pallas as pl
from jax.experimental.pallas import tpu as pltpu
from jax.experimental.pallas import tpu_sc as plsc

_B = 16384
_N_OPS = 100000
_OP_EMB = 64
_LIN = 64
_OUT = _OP_EMB + _LIN


_TCOLS = 6656
_NBLK = 8
_PAIR = _NBLK * _TCOLS
_LROWS = _B // _NBLK
_HI_CLAMP = 15


def _stage_lin_body(lo_ref, hi_ref, st_ref, at_ref, wst_ref, bs_ref,
                    wat_ref, ba_ref, stage_ref, lin_ref):
    stage_ref[...] = jnp.concatenate([lo_ref[...].T, hi_ref[...].T], axis=-1)
    se = lax.dot_general(st_ref[...], wst_ref[...], (((0,), (0,)), ((), ())),
                         preferred_element_type=jnp.float32) + bs_ref[...]
    ae = lax.dot_general(at_ref[...], wat_ref[...], (((0,), (0,)), ((), ())),
                         preferred_element_type=jnp.float32) + ba_ref[...]
    lin_ref[...] = jnp.concatenate([se, ae], axis=-1)


def _stage_and_linears(table_t, shapes_t, attrs_t, w_shape_t, b_shape,
                       w_attr_t, b_attr):
    return pl.pallas_call(
        _stage_lin_body,
        grid=(_NBLK,),
        in_specs=[
            pl.BlockSpec((_OP_EMB, _TCOLS), lambda j: (0, j)),
            pl.BlockSpec((_OP_EMB, _TCOLS),
                         lambda j: (0, jnp.minimum(j + _NBLK, _HI_CLAMP))),
            pl.BlockSpec((shapes_t.shape[0], _LROWS), lambda j: (0, j)),
            pl.BlockSpec((attrs_t.shape[0], _LROWS), lambda j: (0, j)),
            pl.BlockSpec(w_shape_t.shape, lambda j: (0, 0)),
            pl.BlockSpec((1, b_shape.shape[1]), lambda j: (0, 0)),
            pl.BlockSpec(w_attr_t.shape, lambda j: (0, 0)),
            pl.BlockSpec((1, b_attr.shape[1]), lambda j: (0, 0)),
        ],
        out_specs=[
            pl.BlockSpec((_TCOLS, 2 * _OP_EMB), lambda j: (j, 0)),
            pl.BlockSpec((_LROWS, _LIN), lambda j: (j, 0)),
        ],
        out_shape=[
            jax.ShapeDtypeStruct((_PAIR, 2 * _OP_EMB), jnp.float32),
            jax.ShapeDtypeStruct((_B, _LIN), jnp.float32),
        ],
    )(table_t, table_t, shapes_t, attrs_t, w_shape_t, b_shape, w_attr_t,
      b_attr)


_info = plsc.get_sparse_core_info()
_NC, _NS = _info.num_cores, _info.num_subcores
_NW = _NC * _NS
_BPW = _B // _NW
_CH = 128
_NCH = _BPW // _CH
_L = _info.num_lanes

_sc_mesh = plsc.VectorSubcoreMesh(core_axis_name="c", subcore_axis_name="s")


@functools.partial(
    pl.kernel,
    mesh=_sc_mesh,
    out_type=jax.ShapeDtypeStruct((_B, _OUT), jnp.float32),
    scratch_types=[
        pltpu.VMEM((_BPW,), jnp.int32),
        pltpu.VMEM((_BPW,), jnp.int32),
        pltpu.VMEM((_NCH, _CH, 2 * _OP_EMB), jnp.float32),
        pltpu.VMEM((2, _CH, _OUT), jnp.float32),
        pltpu.VMEM((_CH, _LIN), jnp.float32),
        pltpu.SemaphoreType.DMA,
        pltpu.SemaphoreType.DMA,
    ],
)
def _sc_gather(ids_hbm, offs_hbm, table2_hbm, lin_hbm, out_hbm, idx_v,
               off_v, raw_v, cmp_v, lin_v, gsem, osem):
    wid = lax.axis_index("s") * _NC + lax.axis_index("c")
    base = wid * _BPW
    pltpu.sync_copy(ids_hbm.at[pl.ds(base, _BPW)], idx_v)
    pltpu.sync_copy(offs_hbm.at[pl.ds(base, _BPW)], off_v)
    gathers = [
        pltpu.async_copy(table2_hbm.at[idx_v.at[pl.ds(c * _CH, _CH)]],
                         raw_v.at[c], gsem)
        for c in range(_NCH)
    ]
    outs = []
    for c in range(_NCH):
        buf = c % 2
        if c >= 2:
            outs[c - 2].wait()
        pltpu.sync_copy(lin_hbm.at[pl.ds(base + c * _CH, _CH)], lin_v)
        gathers[c].wait()
        def _grp(g, carry, c=c, buf=buf):
            offv = off_v[pl.ds(c * _CH + g * _L, _L)]
            rawg = raw_v.at[c, pl.ds(g * _L, _L)]
            cmpg = cmp_v.at[buf, pl.ds(g * _L, _L)]
            ling = lin_v.at[pl.ds(g * _L, _L)]
            for lane in range(_L):
                o = offv[lane]
                for q in range(_OP_EMB // _L):
                    cmpg[lane, pl.ds(q * _L, _L)] = (
                        rawg[lane, pl.ds(o + q * _L, _L)])
                for q in range(_LIN // _L):
                    cmpg[lane, pl.ds(_OP_EMB + q * _L, _L)] = (
                        ling[lane, pl.ds(q * _L, _L)])
            return carry

        lax.fori_loop(0, _CH // _L, _grp, jnp.int32(0), unroll=2)
        outs.append(pltpu.async_copy(
            cmp_v.at[buf], out_hbm.at[pl.ds(base + c * _CH, _CH)], osem))
    for cp in outs[-2:]:
        cp.wait()


def kernel(op_ids, shapes, attrs, op_table, w_shape, b_shape, w_attr, b_attr):
    table2, lin = _stage_and_linears(
        op_table.T, shapes.T, attrs.T, w_shape.T, b_shape.reshape(1, -1),
        w_attr.T, b_attr.reshape(1, -1))
    hi = op_ids >= _PAIR
    ids2 = jnp.where(hi, op_ids - _PAIR, op_ids)
    offs = jnp.where(hi, _OP_EMB, 0).astype(jnp.int32)
    return _sc_gather(ids2, offs, table2, lin)
